# baseline (speedup 1.0000x reference)
import jax
import jax.numpy as jnp
import numpy as np
from jax import lax
from jax.experimental import pallas as pl
from jax.experimental.pallas import tpu as pltpu

K_CL = 5
EPS = 0.001


def _body(bs_ref, table_ref, w1_ref, b1_ref, w2_ref, b2_ref,
          soft_ref, rank_ref, scores_ref):
    B, rows, col = table_ref.shape
    x2 = table_ref[...].reshape(B * rows, col)
    h1 = jnp.maximum(
        lax.dot_general(x2, w1_ref[...], (((1,), (0,)), ((), ())),
                        preferred_element_type=jnp.float32) + b1_ref[...], 0.0)
    h = jnp.maximum(
        lax.dot_general(h1, w2_ref[...], (((1,), (0,)), ((), ())),
                        preferred_element_type=jnp.float32) + b2_ref[...], 0.0)

    x0 = h[:rows]
    init_idx = np.linspace(0, rows - 1, K_CL).astype(np.int32)
    c0 = jnp.concatenate([x0[int(i):int(i) + 1, :] for i in init_idx], axis=0)
    ones_col = jnp.ones((rows, 1), dtype=jnp.float32)
    kvec = lax.broadcasted_iota(jnp.int32, (1, K_CL), 1)

    def km_body(_, carry):
        c, _lab = carry
        best = jnp.sum((x0 - c[0:1, :]) ** 2, axis=1, keepdims=True)
        lab = jnp.zeros((rows, 1), dtype=jnp.int32)
        for k in range(1, K_CL):
            dk = jnp.sum((x0 - c[k:k + 1, :]) ** 2, axis=1, keepdims=True)
            better = dk < best
            lab = jnp.where(better, k, lab)
            best = jnp.where(better, dk, best)
        onehot = (lab == kvec).astype(jnp.float32)
        counts = lax.dot_general(onehot, ones_col, (((0,), (0,)), ((), ())),
                                 preferred_element_type=jnp.float32)
        csum = lax.dot_general(onehot, x0, (((0,), (0,)), ((), ())),
                               preferred_element_type=jnp.float32)
        return csum / jnp.maximum(counts, 1.0), lab

    _, labels = lax.fori_loop(0, 10, km_body,
                              (c0, jnp.zeros((rows, 1), jnp.int32)))

    onehot = (labels == kvec).astype(jnp.float32)
    counts = lax.dot_general(onehot, ones_col, (((0,), (0,)), ((), ())),
                             preferred_element_type=jnp.float32)
    inv_counts = 1.0 / jnp.maximum(counts, 1.0)
    lab_f = labels.astype(jnp.float32)

    bs_f = bs_ref[0].astype(jnp.float32)
    CH = 256
    nch = rows // CH
    ones_ch = jnp.ones((CH, 1), dtype=jnp.float32)
    tri = (lax.broadcasted_iota(jnp.int32, (CH, CH), 1)
           < lax.broadcasted_iota(jnp.int32, (CH, CH), 0)
           ).astype(jnp.float32)

    for b in range(B):
        hb = h[b * rows:(b + 1) * rows]
        centers = lax.dot_general(onehot, hb, (((0,), (0,)), ((), ())),
                                  preferred_element_type=jnp.float32) * inv_counts
        cdata = lax.dot_general(onehot, centers, (((1,), (0,)), ((), ())),
                                preferred_element_type=jnp.float32,
                                precision=lax.Precision.HIGHEST)
        dist = jnp.mean((hb - cdata) ** 2, axis=1, keepdims=True)
        mn = jnp.min(dist)
        mx = jnp.max(dist)
        sco = (dist - mn) / (mx - mn) + lab_f
        mn2 = jnp.min(sco)
        mx2 = jnp.max(sco)
        sca = (sco - mn2) / (mx2 - mn2) * float(B)
        scores_ref[b] = sco

        s_row = lax.transpose(sco, (1, 0))
        t_row5 = lax.transpose(sca, (1, 0)) * (0.5 / EPS)

        rowS = []
        rowC = []
        colS = []
        colC = []
        for p in range(nch):
            lo = p * CH
            W = rows - lo
            s_blk = sco[lo:lo + CH]
            t_blk5 = sca[lo:lo + CH] * (0.5 / EPS)
            s_rs = s_row[:, lo:]
            t_rs = t_row5[:, lo:]

            sig = jnp.tanh(t_blk5 - t_rs) * 0.5 + 0.5
            lt = (s_rs < s_blk).astype(jnp.float32)
            eq_d = s_rs[:, :CH] == s_blk
            cnt_d = jnp.where(eq_d, tri, lt[:, :CH])

            ones_w = jnp.ones((W, 1), dtype=jnp.float32)
            rowS.append(lax.dot_general(
                sig, ones_w, (((1,), (0,)), ((), ())),
                preferred_element_type=jnp.float32))
            rC = lax.dot_general(cnt_d, ones_ch, (((1,), (0,)), ((), ())),
                                 preferred_element_type=jnp.float32)
            if W > CH:
                rC = rC + lax.dot_general(
                    lt[:, CH:], jnp.ones((W - CH, 1), dtype=jnp.float32),
                    (((1,), (0,)), ((), ())),
                    preferred_element_type=jnp.float32)
                colS.append(lax.dot_general(
                    sig, ones_ch, (((0,), (0,)), ((), ())),
                    preferred_element_type=jnp.float32))
                colC.append(lax.dot_general(
                    lt, ones_ch, (((0,), (0,)), ((), ())),
                    preferred_element_type=jnp.float32))
            rowC.append(rC)

        for p in range(nch):
            lo = p * CH
            softp = rowS[p] + (0.5 + float(lo))
            hardp = rowC[p] + float(lo)
            for q in range(p):
                off = lo - q * CH
                softp = softp - colS[q][off:off + CH]
                hardp = hardp - colC[q][off:off + CH]
            soft_ref[b, lo:lo + CH, :] = softp
            rank_ref[b, lo:lo + CH, :] = (jnp.floor(hardp / bs_f) + 1.0
                                          ).astype(jnp.int32)


def kernel(table, W1, b1, W2, b2, BlockSize, current_epoch):
    B, rows, col = table.shape
    bs = jnp.asarray(BlockSize, jnp.int32).reshape(1)
    soft, rank, scores = pl.pallas_call(
        _body,
        in_specs=[
            pl.BlockSpec(memory_space=pltpu.SMEM),
            pl.BlockSpec(memory_space=pltpu.VMEM),
            pl.BlockSpec(memory_space=pltpu.VMEM),
            pl.BlockSpec(memory_space=pltpu.VMEM),
            pl.BlockSpec(memory_space=pltpu.VMEM),
            pl.BlockSpec(memory_space=pltpu.VMEM),
        ],
        out_shape=(
            jax.ShapeDtypeStruct((B, rows, 1), jnp.float32),
            jax.ShapeDtypeStruct((B, rows, 1), jnp.int32),
            jax.ShapeDtypeStruct((B, rows, 1), jnp.float32),
        ),
    )(bs, table, W1, b1.reshape(1, -1), W2, b2.reshape(1, -1))
    return soft, rank, scores


# speedup vs baseline: 1.1150x; 1.1150x over previous
import jax
import jax.numpy as jnp
import numpy as np
from jax import lax
from jax.experimental import pallas as pl
from jax.experimental.pallas import tpu as pltpu

K_CL = 5
EPS = 0.001


def _body(bs_ref, table_ref, w1_ref, b1_ref, w2_ref, b2_ref,
          soft_ref, rank_ref, scores_ref):
    B, rows, col = table_ref.shape
    x2 = table_ref[...].reshape(B * rows, col)
    h1 = jnp.maximum(
        lax.dot_general(x2, w1_ref[...], (((1,), (0,)), ((), ())),
                        preferred_element_type=jnp.float32) + b1_ref[...], 0.0)
    h = jnp.maximum(
        lax.dot_general(h1, w2_ref[...], (((1,), (0,)), ((), ())),
                        preferred_element_type=jnp.float32) + b2_ref[...], 0.0)

    x0 = h[:rows]
    init_idx = np.linspace(0, rows - 1, K_CL).astype(np.int32)
    c0 = jnp.concatenate([x0[int(i):int(i) + 1, :] for i in init_idx], axis=0)
    ones_col = jnp.ones((rows, 1), dtype=jnp.float32)
    kvec = lax.broadcasted_iota(jnp.int32, (1, K_CL), 1)

    def km_body(_, carry):
        c, _lab = carry
        best = jnp.sum((x0 - c[0:1, :]) ** 2, axis=1, keepdims=True)
        lab = jnp.zeros((rows, 1), dtype=jnp.int32)
        for k in range(1, K_CL):
            dk = jnp.sum((x0 - c[k:k + 1, :]) ** 2, axis=1, keepdims=True)
            better = dk < best
            lab = jnp.where(better, k, lab)
            best = jnp.where(better, dk, best)
        onehot = (lab == kvec).astype(jnp.float32)
        counts = lax.dot_general(onehot, ones_col, (((0,), (0,)), ((), ())),
                                 preferred_element_type=jnp.float32)
        csum = lax.dot_general(onehot, x0, (((0,), (0,)), ((), ())),
                               preferred_element_type=jnp.float32)
        return csum / jnp.maximum(counts, 1.0), lab

    _, labels = lax.fori_loop(0, 10, km_body,
                              (c0, jnp.zeros((rows, 1), jnp.int32)))

    onehot = (labels == kvec).astype(jnp.float32)
    counts = lax.dot_general(onehot, ones_col, (((0,), (0,)), ((), ())),
                             preferred_element_type=jnp.float32)
    inv_counts = 1.0 / jnp.maximum(counts, 1.0)
    lab_f = labels.astype(jnp.float32)

    bs_f = bs_ref[0].astype(jnp.float32)
    CH = 256
    nch = rows // CH
    ones_ch = jnp.ones((CH, 1), dtype=jnp.float32)
    ones_ch_row = jnp.ones((1, CH), dtype=jnp.float32)
    tri = (lax.broadcasted_iota(jnp.int32, (CH, CH), 1)
           < lax.broadcasted_iota(jnp.int32, (CH, CH), 0)
           ).astype(jnp.float32)

    for b in range(B):
        hb = h[b * rows:(b + 1) * rows]
        centers = lax.dot_general(onehot, hb, (((0,), (0,)), ((), ())),
                                  preferred_element_type=jnp.float32) * inv_counts
        cdata = lax.dot_general(onehot, centers, (((1,), (0,)), ((), ())),
                                preferred_element_type=jnp.float32,
                                precision=lax.Precision.HIGHEST)
        dist = jnp.mean((hb - cdata) ** 2, axis=1, keepdims=True)
        mn = jnp.min(dist)
        mx = jnp.max(dist)
        sco = (dist - mn) / (mx - mn) + lab_f
        mn2 = jnp.min(sco)
        mx2 = jnp.max(sco)
        sca = (sco - mn2) / (mx2 - mn2) * float(B)
        scores_ref[b] = sco

        s_row = lax.transpose(sco, (1, 0))
        t_row5 = lax.transpose(sca, (1, 0)) * (0.5 / EPS)

        rowS = []
        rowC = []
        colS = []
        colC = []
        for p in range(nch):
            lo = p * CH
            W = rows - lo
            s_blk = sco[lo:lo + CH]
            t_blk5 = sca[lo:lo + CH] * (0.5 / EPS)
            s_rs = s_row[:, lo:]
            t_rs = t_row5[:, lo:]

            sig = jnp.tanh(t_blk5 - t_rs) * 0.5 + 0.5
            lt = (s_rs < s_blk).astype(jnp.float32)
            eq_d = s_rs[:, :CH] == s_blk
            cnt_d = jnp.where(eq_d, tri, lt[:, :CH])

            ones_w = jnp.ones((W, 1), dtype=jnp.float32)
            rowS.append(lax.dot_general(
                sig, ones_w, (((1,), (0,)), ((), ())),
                preferred_element_type=jnp.float32))
            rC = lax.dot_general(cnt_d, ones_ch, (((1,), (0,)), ((), ())),
                                 preferred_element_type=jnp.float32)
            if W > CH:
                rC = rC + lax.dot_general(
                    lt[:, CH:], jnp.ones((W - CH, 1), dtype=jnp.float32),
                    (((1,), (0,)), ((), ())),
                    preferred_element_type=jnp.float32)
                colS.append(lax.dot_general(
                    ones_ch_row, sig, (((1,), (0,)), ((), ())),
                    preferred_element_type=jnp.float32))
                colC.append(lax.dot_general(
                    ones_ch_row, lt, (((1,), (0,)), ((), ())),
                    preferred_element_type=jnp.float32))
            rowC.append(rC)

        for p in range(nch):
            lo = p * CH
            softp = rowS[p] + (0.5 + float(lo))
            hardp = rowC[p] + float(lo)
            if p > 0:
                csum = colS[0][:, lo:lo + CH]
                ccnt = colC[0][:, lo:lo + CH]
                for q in range(1, p):
                    off = lo - q * CH
                    csum = csum + colS[q][:, off:off + CH]
                    ccnt = ccnt + colC[q][:, off:off + CH]
                softp = softp - lax.transpose(csum, (1, 0))
                hardp = hardp - lax.transpose(ccnt, (1, 0))
            soft_ref[b, lo:lo + CH, :] = softp
            rank_ref[b, lo:lo + CH, :] = (jnp.floor(hardp / bs_f) + 1.0
                                          ).astype(jnp.int32)


def kernel(table, W1, b1, W2, b2, BlockSize, current_epoch):
    B, rows, col = table.shape
    bs = jnp.asarray(BlockSize, jnp.int32).reshape(1)
    soft, rank, scores = pl.pallas_call(
        _body,
        in_specs=[
            pl.BlockSpec(memory_space=pltpu.SMEM),
            pl.BlockSpec(memory_space=pltpu.VMEM),
            pl.BlockSpec(memory_space=pltpu.VMEM),
            pl.BlockSpec(memory_space=pltpu.VMEM),
            pl.BlockSpec(memory_space=pltpu.VMEM),
            pl.BlockSpec(memory_space=pltpu.VMEM),
        ],
        out_shape=(
            jax.ShapeDtypeStruct((B, rows, 1), jnp.float32),
            jax.ShapeDtypeStruct((B, rows, 1), jnp.int32),
            jax.ShapeDtypeStruct((B, rows, 1), jnp.float32),
        ),
    )(bs, table, W1, b1.reshape(1, -1), W2, b2.reshape(1, -1))
    return soft, rank, scores
